# baseline (speedup 1.0000x reference)
import functools

import jax
import jax.numpy as jnp
from jax import lax
from jax.experimental import pallas as pl
from jax.experimental.pallas import tpu as pltpu
from jax.experimental.pallas import tpu_sc as plsc

_EMB = 64
_SCALE = _EMB ** (-0.5)
_NW = 32
_LANES = 16
_NBUF = 4


def _sc_embed(x, table, rows_per_w, n_halves):
    B, T = x.shape
    half = T // n_halves
    n_chunks = rows_per_w * n_halves
    mesh = plsc.VectorSubcoreMesh(core_axis_name="c", subcore_axis_name="s")
    n_main = n_chunks - _NBUF

    @functools.partial(
        pl.kernel,
        mesh=mesh,
        compiler_params=pltpu.CompilerParams(use_tc_tiling_on_sc=False),
        out_type=jax.ShapeDtypeStruct((B, T, _EMB), jnp.float32),
        scratch_types=[
            pltpu.VMEM((rows_per_w, T), jnp.int32),
            pltpu.VMEM((_NBUF, half, _EMB), jnp.float32),
        ]
        + [pltpu.SemaphoreType.DMA] * (2 * _NBUF),
    )
    def k(x_hbm, table_hbm, out_hbm, idx_v, rows_v, *sems):
        g_sem = sems[:_NBUF]
        o_sem = sems[_NBUF:]
        wid = lax.axis_index("s") * 2 + lax.axis_index("c")
        row0 = wid * rows_per_w
        pltpu.sync_copy(x_hbm.at[pl.ds(row0, rows_per_w)], idx_v)

        def start_gather(c, b):
            r = c // n_halves
            h = c % n_halves
            pltpu.async_copy(
                table_hbm.at[idx_v.at[r, pl.ds(h * half, half)]],
                rows_v.at[b], g_sem[b])

        def wait_gather(b):
            pltpu.make_async_copy(table_hbm.at[pl.ds(0, half)],
                                  rows_v.at[b], g_sem[b]).wait()

        def scale(b):
            @plsc.parallel_loop(0, half, step=1, unroll=4)
            def _scale_row(r):
                for kk in range(_EMB // _LANES):
                    sl = pl.ds(kk * _LANES, _LANES)
                    rows_v[b, r, sl] = rows_v[b, r, sl] * _SCALE

        def start_out(c, b):
            r = c // n_halves
            h = c % n_halves
            pltpu.async_copy(
                rows_v.at[b],
                out_hbm.at[row0 + r, pl.ds(h * half, half)],
                o_sem[b])

        def wait_out(b):
            pltpu.make_async_copy(rows_v.at[b],
                                  out_hbm.at[0, pl.ds(0, half)],
                                  o_sem[b]).wait()

        for b in range(_NBUF):
            start_gather(b, b)

        def main_body(g, carry):
            c0 = g * _NBUF
            for b in range(_NBUF):
                c = c0 + b
                wait_gather(b)
                scale(b)
                start_out(c, b)
                wait_out(b)
                start_gather(c + _NBUF, b)
            return carry

        lax.fori_loop(0, n_main // _NBUF, main_body, 0)

        for b in range(_NBUF):
            c = n_main + b
            wait_gather(b)
            scale(b)
            start_out(c, b)
        for b in range(_NBUF):
            wait_out(b)

    return k(x, table)


def kernel(x, table):
    B, T = x.shape
    rows_per_w = B // _NW
    n_halves = 5
    return _sc_embed(x.astype(jnp.int32), table, rows_per_w, n_halves)


# speedup vs baseline: 1.2804x; 1.2804x over previous
import functools

import jax
import jax.numpy as jnp
from jax import lax
from jax.experimental import pallas as pl
from jax.experimental.pallas import tpu as pltpu
from jax.experimental.pallas import tpu_sc as plsc

_EMB = 64
_SCALE = _EMB ** (-0.5)
_NW = 32
_LANES = 16
_NBUF = 4
_CHUNK = 128


def _sc_embed(x2d, table_pad):
    n_rows = x2d.shape[0]
    n_chunks = n_rows // _NW
    total = n_rows * _CHUNK
    mesh = plsc.VectorSubcoreMesh(core_axis_name="c", subcore_axis_name="s")
    n_main = n_chunks - _NBUF

    @functools.partial(
        pl.kernel,
        mesh=mesh,
        out_type=jax.ShapeDtypeStruct((total, 128), jnp.float32),
        scratch_types=[
            pltpu.VMEM((n_chunks, _CHUNK), jnp.int32),
            pltpu.VMEM((_NBUF, _CHUNK, 128), jnp.float32),
        ]
        + [pltpu.SemaphoreType.DMA] * (2 * _NBUF),
    )
    def k(x_hbm, table_hbm, out_hbm, idx_v, rows_v, *sems):
        g_sem = sems[:_NBUF]
        o_sem = sems[_NBUF:]
        wid = lax.axis_index("s") * 2 + lax.axis_index("c")
        row0 = wid * n_chunks
        pltpu.sync_copy(x_hbm.at[pl.ds(row0, n_chunks)], idx_v)
        out0 = wid * n_chunks * _CHUNK

        def start_gather(c, b):
            pltpu.async_copy(table_hbm.at[idx_v.at[c]], rows_v.at[b],
                             g_sem[b])

        def wait_gather(b):
            pltpu.make_async_copy(table_hbm.at[pl.ds(0, _CHUNK)],
                                  rows_v.at[b], g_sem[b]).wait()

        def scale(b):
            @plsc.parallel_loop(0, _CHUNK, step=1, unroll=4)
            def _scale_row(r):
                for kk in range(_EMB // _LANES):
                    sl = pl.ds(kk * _LANES, _LANES)
                    rows_v[b, r, sl] = rows_v[b, r, sl] * _SCALE

        def start_out(c, b):
            pltpu.async_copy(rows_v.at[b],
                             out_hbm.at[pl.ds(out0 + c * _CHUNK, _CHUNK)],
                             o_sem[b])

        def wait_out(b):
            pltpu.make_async_copy(rows_v.at[b],
                                  out_hbm.at[pl.ds(0, _CHUNK)],
                                  o_sem[b]).wait()

        for b in range(_NBUF):
            start_gather(b, b)
        for b in range(_NBUF):
            wait_gather(b)
            scale(b)
            start_out(b, b)
            start_gather(b + _NBUF, b)

        def main_body(g, carry):
            c0 = g * _NBUF
            for b in range(_NBUF):
                c = c0 + b
                wait_gather(b)
                wait_out(b)
                scale(b)
                start_out(c, b)
                start_gather(c + _NBUF, b)
            return carry

        lax.fori_loop(1, n_main // _NBUF, main_body, 0)

        for b in range(_NBUF):
            c = n_main + b
            wait_gather(b)
            wait_out(b)
            scale(b)
            start_out(c, b)
        for b in range(_NBUF):
            wait_out(b)

    return k(x2d, table_pad)


def kernel(x, table):
    B, T = x.shape
    n_tok = B * T
    x2d = x.reshape(n_tok // _CHUNK, _CHUNK).astype(jnp.int32)
    table_pad = jnp.pad(table, ((0, 0), (0, 128 - _EMB)))
    out = _sc_embed(x2d, table_pad)
    return out[:, :_EMB].reshape(B, T, _EMB)
